# baseline (speedup 1.0000x reference)
import jax
import jax.numpy as jnp
from jax import lax
from jax.experimental import pallas as pl
from jax.experimental.pallas import tpu as pltpu
from jax.experimental.pallas import tpu_sc as plsc

_R = 131072
_W = 128
_NC = 2
_NS = 16
_NW = _NC * _NS
_CHUNK = 128
_NCHUNK = _R // (_NW * _CHUNK)


def _sc_flip(x_hbm, o_hbm, idx0, idx1, buf0, buf1, sg0, sg1, so0, so1):
    wid = lax.axis_index("s") * _NC + lax.axis_index("c")
    base = wid * _NCHUNK
    lane = lax.iota(jnp.int32, 16)
    idx = (idx0, idx1)
    buf = (buf0, buf1)
    sg = (sg0, sg1)
    so = (so0, so1)

    def fire_gather(c):
        b = c % 2
        slab = base + c
        top = slab * _CHUNK + (_CHUNK - 1)
        for j in range(_CHUNK // 16):
            idx[b][pl.ds(j * 16, 16)] = top - j * 16 - lane
        pltpu.make_async_copy(x_hbm.at[idx[b]], buf[b], sg[b]).start()

    fire_gather(0)
    for c in range(_NCHUNK):
        b = c % 2
        if c + 1 < _NCHUNK:
            if c >= 1:
                pltpu.make_async_copy(
                    buf[(c + 1) % 2],
                    o_hbm.at[pl.ds((base + c - 1) * _CHUNK, _CHUNK)],
                    so[(c + 1) % 2],
                ).wait()
            fire_gather(c + 1)
        pltpu.make_async_copy(x_hbm.at[idx[b]], buf[b], sg[b]).wait()
        pltpu.make_async_copy(
            buf[b], o_hbm.at[pl.ds((base + c) * _CHUNK, _CHUNK)], so[b]
        ).start()
    for c in (_NCHUNK - 2, _NCHUNK - 1):
        b = c % 2
        pltpu.make_async_copy(
            buf[b], o_hbm.at[pl.ds((base + c) * _CHUNK, _CHUNK)], so[b]
        ).wait()


def kernel(x):
    B, C, D, H, W = x.shape
    xr = x.reshape(B * C * D * H, W)
    mesh = plsc.VectorSubcoreMesh(core_axis_name="c", subcore_axis_name="s")
    k = pl.kernel(
        _sc_flip,
        mesh=mesh,
        out_type=jax.ShapeDtypeStruct((_R, _W), jnp.float32),
        scratch_types=[
            pltpu.VMEM((_CHUNK,), jnp.int32),
            pltpu.VMEM((_CHUNK,), jnp.int32),
            pltpu.VMEM((_CHUNK, _W), jnp.float32),
            pltpu.VMEM((_CHUNK, _W), jnp.float32),
            pltpu.SemaphoreType.DMA,
            pltpu.SemaphoreType.DMA,
            pltpu.SemaphoreType.DMA,
            pltpu.SemaphoreType.DMA,
        ],
    )
    out = k(xr)
    return out.reshape(B, C, D, H, W)


# speedup vs baseline: 1.2469x; 1.2469x over previous
import jax
import jax.numpy as jnp
from jax.experimental import pallas as pl

_HB = 8
_NG = 16


def _flip_body(x_ref, o_ref):
    for g in range(_NG):
        blk = x_ref[:, (_NG - 1 - g) * _HB : (_NG - g) * _HB, :]
        o_ref[:, g * _HB : (g + 1) * _HB, :] = jnp.concatenate(
            [blk[:, i : i + 1, :] for i in reversed(range(_HB))], axis=1
        )


def kernel(x):
    B, C, D, H, W = x.shape
    L = B * C * D
    xr = x.reshape(L, H, W)
    Lb = 32
    out = pl.pallas_call(
        _flip_body,
        grid=(L // Lb,),
        in_specs=[pl.BlockSpec((Lb, H, W), lambda l: (l, 0, 0))],
        out_specs=pl.BlockSpec((Lb, H, W), lambda l: (l, 0, 0)),
        out_shape=jax.ShapeDtypeStruct((L, H, W), x.dtype),
    )(xr)
    return out.reshape(B, C, D, H, W)


# speedup vs baseline: 1.4795x; 1.1865x over previous
import jax
import jax.numpy as jnp
from jax.experimental import pallas as pl

_HB = 8
_NG = 16


def _flip_body(x_ref, o_ref):
    for g in range(_NG):
        blk = x_ref[:, (_NG - 1 - g) * _HB : (_NG - g) * _HB, :]
        o_ref[:, g * _HB : (g + 1) * _HB, :] = jnp.concatenate(
            [blk[:, i : i + 1, :] for i in reversed(range(_HB))], axis=1
        )


def kernel(x):
    B, C, D, H, W = x.shape
    L = B * C * D
    xr = x.reshape(L, H, W)
    Lb = 128
    out = pl.pallas_call(
        _flip_body,
        grid=(L // Lb,),
        in_specs=[pl.BlockSpec((Lb, H, W), lambda l: (l, 0, 0))],
        out_specs=pl.BlockSpec((Lb, H, W), lambda l: (l, 0, 0)),
        out_shape=jax.ShapeDtypeStruct((L, H, W), x.dtype),
    )(xr)
    return out.reshape(B, C, D, H, W)
